# baseline (speedup 1.0000x reference)
import functools

import jax
import jax.numpy as jnp
from jax import lax
from jax.experimental import pallas as pl
from jax.experimental.pallas import tpu as pltpu
from jax.experimental.pallas import tpu_sc as plsc

_B = 16384
_D = 64
_NT = 1_000_000
_NC = 2
_NS = 16
_NW = _NC * _NS
_L = 16
_WROWS = 512
_NWIN = 1954
_NMAIN = 1953
_TAIL0 = _NMAIN * _WROWS
_TAILW = (_NMAIN & (_NW - 1))
_TPW = 62
_ROWS_OUT = _B + 8
_DUMMY = _B
_HCAP = _B + _L

_mesh = plsc.VectorSubcoreMesh(core_axis_name="c", subcore_axis_name="s")

_CPT = pltpu.CompilerParams(needs_layout_passes=False, use_tc_tiling_on_sc=True)
_CPF = pltpu.CompilerParams(needs_layout_passes=False, use_tc_tiling_on_sc=False)

_iota = lambda: lax.iota(jnp.int32, _L)


def _popcount(m):
    c = plsc.all_reduce_population_count(m)
    if getattr(c, "ndim", 0):
        c = c[0]
    return c


def _scan_hits(idsv, hitb, wid):
    iota = _iota()

    def sbody(g, cnt):
        b16 = g * _L + iota
        u16 = idsv[pl.ds(g * _L, _L)]
        m = jnp.bitwise_and(lax.shift_right_logical(u16, 9), _NW - 1) == wid
        plsc.store_compressed(hitb.at[pl.ds(cnt, _L)], b16, mask=m)
        return cnt + _popcount(m)

    cnt = lax.fori_loop(0, _B // _L, sbody, jnp.int32(0))
    hitb[pl.ds(cnt, _L)] = jnp.zeros((_L,), jnp.int32)
    return cnt


def _window_hits(idsv, hitb, cnt, wb, wd):
    iota = _iota()

    def wbody(g, wcnt):
        b16 = hitb[pl.ds(g * _L, _L)]
        u16 = plsc.load_gather(idsv, [b16])
        valid = (g * _L + iota) < cnt
        m = jnp.logical_and(valid, lax.shift_right_logical(u16, 9) == wd)
        plsc.store_compressed(wb.at[pl.ds(wcnt, _L)], b16, mask=m)
        return wcnt + _popcount(m)

    ngrp = lax.div(cnt + (_L - 1), jnp.int32(_L))
    wcnt = lax.fori_loop(0, ngrp, wbody, jnp.int32(0))
    wb[pl.ds(wcnt, _L)] = jnp.zeros((_L,), jnp.int32)
    return wcnt


def _fire_window(tabT, win, start, sem):
    return [
        pltpu.async_copy(
            tabT.at[pl.ds(a * 8, 8), pl.ds(start, _WROWS)], win.at[a], sem)
        for a in range(8)
    ]


@functools.partial(
    pl.kernel,
    out_type=jax.ShapeDtypeStruct((_ROWS_OUT, 2 * _D), jnp.float32),
    mesh=_mesh,
    compiler_params=_CPT,
    scratch_types=[
        pltpu.VMEM((_B,), jnp.int32),
        pltpu.VMEM((_HCAP,), jnp.int32),
        pltpu.VMEM((_HCAP,), jnp.int32),
        pltpu.VMEM((8, 8, _WROWS), jnp.float32),
        pltpu.VMEM((64, 2 * _D), jnp.float32),
        pltpu.VMEM((_L, 2 * _D), jnp.float32),
        pltpu.VMEM((_L,), jnp.int32),
        pltpu.SemaphoreType.DMA,
    ],
)
def _scan_item(iid_hbm, tabT_hbm, tail_hbm, rows_out,
               idsv, hitb, wb, win, tailv, stg, bidx, sem):
    wid = lax.axis_index("s") * _NC + lax.axis_index("c")
    iota = _iota()
    pltpu.sync_copy(iid_hbm, idsv)
    pltpu.sync_copy(tail_hbm, tailv)
    cnt = _scan_hits(idsv, hitb, wid)

    def do_window(wd, start, tail):
        wcnt = _window_hits(idsv, hitb, cnt, wb, wd)

        def gbody(g, carry):
            b16 = wb[pl.ds(g * _L, _L)]
            sel = (g * _L + iota) < wcnt
            bsel = jnp.where(sel, b16, _DUMMY)
            u16 = plsc.load_gather(idsv, [b16])
            lrow = jnp.where(sel, u16 - start, 0)
            for a in range(8):
                for dn in range(8):
                    d = a * 8 + dn
                    if tail:
                        vals = plsc.load_gather(
                            tailv, [lrow, jnp.full((_L,), d, jnp.int32)])
                    else:
                        vals = plsc.load_gather(
                            win.at[a],
                            [jnp.full((_L,), dn, jnp.int32), lrow])
                    plsc.store_scatter(
                        stg, [iota, jnp.full((_L,), d, jnp.int32)], vals)
            bidx[...] = bsel
            pltpu.async_copy(stg, rows_out.at[bidx], sem).wait()
            return carry

        lax.fori_loop(0, lax.div(wcnt + (_L - 1), jnp.int32(_L)), gbody, 0)

    def wloop(t, carry):
        wd = wid + _NW * t

        @pl.when(wd < _NMAIN)
        def _():
            start = pl.multiple_of(wd * _WROWS, _WROWS)
            cps = _fire_window(tabT_hbm, win, start, sem)
            for cp in cps:
                cp.wait()
            do_window(wd, start, False)

        return carry

    lax.fori_loop(0, _TPW, wloop, 0)

    @pl.when(wid == _TAILW)
    def _():
        do_window(jnp.int32(_NMAIN), jnp.int32(_TAIL0), True)


@functools.partial(
    pl.kernel,
    out_type=jax.ShapeDtypeStruct((_ROWS_OUT,), jnp.float32),
    mesh=_mesh,
    compiler_params=_CPT,
    scratch_types=[
        pltpu.VMEM((_B,), jnp.int32),
        pltpu.VMEM((_HCAP,), jnp.int32),
        pltpu.VMEM((_HCAP,), jnp.int32),
        pltpu.VMEM((8, 8, _WROWS), jnp.float32),
        pltpu.VMEM((64, 2 * _D), jnp.float32),
        pltpu.VMEM((_L, 2 * _D), jnp.float32),
        pltpu.VMEM((_L,), jnp.int32),
        pltpu.VMEM((_L,), jnp.float32),
        pltpu.SemaphoreType.DMA,
    ],
)
def _scan_user(uid_hbm, tabT_hbm, tail_hbm, itemrows_hbm, dots_out,
               idsv, hitb, wb, win, tailv, irow, bidx, dotv, sem):
    wid = lax.axis_index("s") * _NC + lax.axis_index("c")
    iota = _iota()
    pltpu.sync_copy(uid_hbm, idsv)
    pltpu.sync_copy(tail_hbm, tailv)
    cnt = _scan_hits(idsv, hitb, wid)

    def do_window(wd, start, tail):
        wcnt = _window_hits(idsv, hitb, cnt, wb, wd)

        def gbody(g, carry):
            b16 = wb[pl.ds(g * _L, _L)]
            sel = (g * _L + iota) < wcnt
            bsel = jnp.where(sel, b16, _DUMMY)
            u16 = plsc.load_gather(idsv, [b16])
            lrow = jnp.where(sel, u16 - start, 0)
            bidx[...] = bsel
            pltpu.async_copy(itemrows_hbm.at[bidx], irow, sem).wait()
            acc = jnp.zeros((_L,), jnp.float32)
            for a in range(8):
                for dn in range(8):
                    d = a * 8 + dn
                    if tail:
                        ug = plsc.load_gather(
                            tailv, [lrow, jnp.full((_L,), d, jnp.int32)])
                    else:
                        ug = plsc.load_gather(
                            win.at[a],
                            [jnp.full((_L,), dn, jnp.int32), lrow])
                    ig = plsc.load_gather(
                        irow, [iota, jnp.full((_L,), d, jnp.int32)])
                    acc = acc + ug * ig
            dotv[...] = acc
            pltpu.async_copy(dotv, dots_out.at[bidx], sem).wait()
            return carry

        lax.fori_loop(0, lax.div(wcnt + (_L - 1), jnp.int32(_L)), gbody, 0)

    def wloop(t, carry):
        wd = wid + _NW * t

        @pl.when(wd < _NMAIN)
        def _():
            start = pl.multiple_of(wd * _WROWS, _WROWS)
            cps = _fire_window(tabT_hbm, win, start, sem)
            for cp in cps:
                cp.wait()
            do_window(wd, start, False)

        return carry

    lax.fori_loop(0, _TPW, wloop, 0)

    @pl.when(wid == _TAILW)
    def _():
        do_window(jnp.int32(_NMAIN), jnp.int32(_TAIL0), True)


_BPW = _B // _NW
_CHUNK = 128
_NCHUNK = _BPW // _CHUNK
_GPC = _CHUNK // _L


@functools.partial(
    pl.kernel,
    out_type=jax.ShapeDtypeStruct((_NW, _BPW), jnp.float32),
    mesh=_mesh,
    compiler_params=_CPF,
    scratch_types=[
        pltpu.VMEM((_NCHUNK, _CHUNK), jnp.int32),
        pltpu.VMEM((_NCHUNK, _CHUNK), jnp.int32),
        pltpu.VMEM((_NCHUNK, _CHUNK), jnp.int32),
        pltpu.VMEM((_NCHUNK, _CHUNK), jnp.int32),
        pltpu.VMEM((2, _CHUNK, 8), jnp.float32),
        pltpu.VMEM((2, _CHUNK, 8), jnp.float32),
        pltpu.VMEM((_BPW,), jnp.float32),
        pltpu.SemaphoreType.DMA,
    ],
)
def _bias_sum(uid_hbm, iid_hbm, ubias_hbm, ibias_hbm, out_hbm,
              uidx, iidx, ubdx, ibdx, ubb, ibb, outv, sem):
    wid = lax.axis_index("s") * _NC + lax.axis_index("c")
    pltpu.sync_copy(uid_hbm.at[wid], uidx)
    pltpu.sync_copy(iid_hbm.at[wid], iidx)
    for c in range(_NCHUNK):
        for j in range(_GPC):
            sl = pl.ds(j * _L, _L)
            ubdx[c, sl] = lax.shift_right_logical(uidx[c, sl], 3)
            ibdx[c, sl] = lax.shift_right_logical(iidx[c, sl], 3)

    def fire(c):
        slot = c % 2
        return [
            pltpu.async_copy(ubias_hbm.at[ubdx.at[c]], ubb.at[slot], sem),
            pltpu.async_copy(ibias_hbm.at[ibdx.at[c]], ibb.at[slot], sem),
        ]

    iota = _iota()
    pending = fire(0)
    for c in range(_NCHUNK):
        for cp in pending:
            cp.wait()
        if c + 1 < _NCHUNK:
            pending = fire(c + 1)
        slot = c % 2
        ubb_c = ubb.at[slot]
        ibb_c = ibb.at[slot]

        def body(g, carry, c=c, ubb_c=ubb_c, ibb_c=ibb_c):
            rowk = g * _L + iota
            su = uidx[c, pl.ds(g * _L, _L)]
            si = iidx[c, pl.ds(g * _L, _L)]
            bsum = (plsc.load_gather(ubb_c, [rowk, jnp.bitwise_and(su, 7)])
                    + plsc.load_gather(ibb_c, [rowk, jnp.bitwise_and(si, 7)]))
            outv[pl.ds(c * _CHUNK + g * _L, _L)] = bsum
            return carry

        lax.fori_loop(0, _GPC, body, 0)

    pltpu.sync_copy(outv, out_hbm.at[wid])


def kernel(user_ids, item_ids, user_emb, item_emb, user_bias, item_bias):
    uid = user_ids.astype(jnp.int32)
    iid = item_ids.astype(jnp.int32)
    tail_u = jnp.pad(user_emb[_TAIL0:], ((0, 0), (0, _D)))
    tail_i = jnp.pad(item_emb[_TAIL0:], ((0, 0), (0, _D)))
    item_rows = _scan_item(iid, item_emb.T, tail_i)
    dots = _scan_user(uid, user_emb.T, tail_u, item_rows)
    bias = _bias_sum(uid.reshape(_NW, _NCHUNK, _CHUNK),
                     iid.reshape(_NW, _NCHUNK, _CHUNK),
                     user_bias.reshape(-1, 8), item_bias.reshape(-1, 8))
    return dots[:_B] + bias.reshape(_B)


# speedup vs baseline: 2.8797x; 2.8797x over previous
import functools

import jax
import jax.numpy as jnp
from jax import lax
from jax.experimental import pallas as pl
from jax.experimental.pallas import tpu as pltpu
from jax.experimental.pallas import tpu_sc as plsc

_B = 16384
_D = 64
_NC = 2
_NS = 16
_NW = _NC * _NS
_BPW = _B // _NW
_CHUNK = 128
_NCHUNK = _BPW // _CHUNK
_L = 16
_GPC = _CHUNK // _L

_mesh = plsc.VectorSubcoreMesh(core_axis_name="c", subcore_axis_name="s")


@functools.partial(
    pl.kernel,
    out_type=jax.ShapeDtypeStruct((_NW, _BPW), jnp.float32),
    mesh=_mesh,
    compiler_params=pltpu.CompilerParams(
        needs_layout_passes=False,
        use_tc_tiling_on_sc=False,
    ),
    scratch_types=[
        pltpu.VMEM((_NCHUNK, _CHUNK), jnp.int32),
        pltpu.VMEM((_NCHUNK, _CHUNK), jnp.int32),
        pltpu.VMEM((_NCHUNK, _CHUNK), jnp.int32),
        pltpu.VMEM((_NCHUNK, _CHUNK), jnp.int32),
        pltpu.VMEM((_NCHUNK, _CHUNK), jnp.int32),
        pltpu.VMEM((_NCHUNK, _CHUNK), jnp.int32),
        pltpu.VMEM((2, _CHUNK, 2 * _D), jnp.float32),
        pltpu.VMEM((2, _CHUNK, 2 * _D), jnp.float32),
        pltpu.VMEM((2, _CHUNK, 8), jnp.float32),
        pltpu.VMEM((2, _CHUNK, 8), jnp.float32),
        pltpu.VMEM((_BPW,), jnp.float32),
        pltpu.SemaphoreType.DMA,
    ],
)
def _sc_kernel(uid_hbm, iid_hbm, uemb_hbm, iemb_hbm, ubias_hbm, ibias_hbm,
               out_hbm, uidx, iidx, updx, ipdx, ubdx, ibdx,
               ubuf, ibuf, ubb, ibb, outv, sem):
    wid = lax.axis_index("s") * _NC + lax.axis_index("c")

    pltpu.sync_copy(uid_hbm.at[wid], uidx)
    pltpu.sync_copy(iid_hbm.at[wid], iidx)

    for c in range(_NCHUNK):
        for j in range(_GPC):
            sl = pl.ds(j * _L, _L)
            u = uidx[c, sl]
            i = iidx[c, sl]
            updx[c, sl] = lax.shift_right_logical(u, 1)
            ipdx[c, sl] = lax.shift_right_logical(i, 1)
            ubdx[c, sl] = lax.shift_right_logical(u, 3)
            ibdx[c, sl] = lax.shift_right_logical(i, 3)

    def fire(c):
        slot = c % 2
        return [
            pltpu.async_copy(uemb_hbm.at[updx.at[c]], ubuf.at[slot], sem),
            pltpu.async_copy(iemb_hbm.at[ipdx.at[c]], ibuf.at[slot], sem),
            pltpu.async_copy(ubias_hbm.at[ubdx.at[c]], ubb.at[slot], sem),
            pltpu.async_copy(ibias_hbm.at[ibdx.at[c]], ibb.at[slot], sem),
        ]

    iota = lax.iota(jnp.int32, _L)

    pending = fire(0)
    for c in range(_NCHUNK):
        for cp in pending:
            cp.wait()
        if c + 1 < _NCHUNK:
            pending = fire(c + 1)
        slot = c % 2
        ub_c = ubuf.at[slot]
        ib_c = ibuf.at[slot]
        ubb_c = ubb.at[slot]
        ibb_c = ibb.at[slot]

        def body(g, carry, c=c, ub_c=ub_c, ib_c=ib_c, ubb_c=ubb_c, ibb_c=ibb_c):
            rowk = g * _L + iota
            su = uidx[c, pl.ds(g * _L, _L)]
            si = iidx[c, pl.ds(g * _L, _L)]
            colu = lax.shift_left(jnp.bitwise_and(su, 1), 6)
            coli = lax.shift_left(jnp.bitwise_and(si, 1), 6)
            acc = (plsc.load_gather(ubb_c, [rowk, jnp.bitwise_and(su, 7)])
                   + plsc.load_gather(ibb_c, [rowk, jnp.bitwise_and(si, 7)]))
            for d in range(_D):
                pu = plsc.load_gather(ub_c, [rowk, colu + d])
                pi = plsc.load_gather(ib_c, [rowk, coli + d])
                acc = acc + pu * pi
            outv[pl.ds(c * _CHUNK + g * _L, _L)] = acc
            return carry

        lax.fori_loop(0, _GPC, body, 0)

    pltpu.sync_copy(outv, out_hbm.at[wid])


def kernel(user_ids, item_ids, user_emb, item_emb, user_bias, item_bias):
    uid = user_ids.astype(jnp.int32).reshape(_NW, _NCHUNK, _CHUNK)
    iid = item_ids.astype(jnp.int32).reshape(_NW, _NCHUNK, _CHUNK)
    ue = user_emb.reshape(-1, 2 * _D)
    ie = item_emb.reshape(-1, 2 * _D)
    ub = user_bias.reshape(-1, 8)
    ib = item_bias.reshape(-1, 8)
    out = _sc_kernel(uid, iid, ue, ie, ub, ib)
    return out.reshape(_B)


# speedup vs baseline: 3.0762x; 1.0682x over previous
import functools

import jax
import jax.numpy as jnp
from jax import lax
from jax.experimental import pallas as pl
from jax.experimental.pallas import tpu as pltpu
from jax.experimental.pallas import tpu_sc as plsc

_B = 16384
_D = 64
_NC = 2
_NS = 16
_NW = _NC * _NS
_BPW = _B // _NW
_CHUNK = 128
_NCHUNK = _BPW // _CHUNK
_L = 16
_GPC = _CHUNK // _L

_mesh = plsc.VectorSubcoreMesh(core_axis_name="c", subcore_axis_name="s")


@functools.partial(
    pl.kernel,
    out_type=jax.ShapeDtypeStruct((_NW, _BPW), jnp.float32),
    mesh=_mesh,
    compiler_params=pltpu.CompilerParams(
        needs_layout_passes=False,
        use_tc_tiling_on_sc=False,
    ),
    scratch_types=[
        pltpu.VMEM((_NCHUNK, _CHUNK), jnp.int32),
        pltpu.VMEM((_NCHUNK, _CHUNK), jnp.int32),
        pltpu.VMEM((_NCHUNK, _CHUNK), jnp.int32),
        pltpu.VMEM((_NCHUNK, _CHUNK), jnp.int32),
        pltpu.VMEM((2, _CHUNK, 2 * _D), jnp.float32),
        pltpu.VMEM((2, _CHUNK, 2 * _D), jnp.float32),
        pltpu.VMEM((2, _CHUNK, 8), jnp.float32),
        pltpu.VMEM((2, _CHUNK, 8), jnp.float32),
        pltpu.VMEM((_BPW,), jnp.float32),
        pltpu.SemaphoreType.DMA,
    ],
)
def _sc_kernel(uid_hbm, iid_hbm, uemb_hbm, iemb_hbm, ubias_hbm, ibias_hbm,
               out_hbm, uidx, iidx, ubdx, ibdx,
               ubuf, ibuf, ubb, ibb, outv, sem):
    wid = lax.axis_index("s") * _NC + lax.axis_index("c")

    pltpu.sync_copy(uid_hbm.at[wid], uidx)
    pltpu.sync_copy(iid_hbm.at[wid], iidx)

    for c in range(_NCHUNK):
        for j in range(_GPC):
            sl = pl.ds(j * _L, _L)
            ubdx[c, sl] = lax.shift_right_logical(uidx[c, sl], 3)
            ibdx[c, sl] = lax.shift_right_logical(iidx[c, sl], 3)

    def fire(c):
        slot = c % 2
        return [
            pltpu.async_copy(uemb_hbm.at[uidx.at[c]], ubuf.at[slot], sem),
            pltpu.async_copy(iemb_hbm.at[iidx.at[c]], ibuf.at[slot], sem),
            pltpu.async_copy(ubias_hbm.at[ubdx.at[c]], ubb.at[slot], sem),
            pltpu.async_copy(ibias_hbm.at[ibdx.at[c]], ibb.at[slot], sem),
        ]

    iota = lax.iota(jnp.int32, _L)

    pending = fire(0)
    for c in range(_NCHUNK):
        for cp in pending:
            cp.wait()
        if c + 1 < _NCHUNK:
            pending = fire(c + 1)
        slot = c % 2
        ub_c = ubuf.at[slot]
        ib_c = ibuf.at[slot]
        ubb_c = ubb.at[slot]
        ibb_c = ibb.at[slot]

        def body(g, carry, c=c, ub_c=ub_c, ib_c=ib_c, ubb_c=ubb_c, ibb_c=ibb_c):
            rowk = g * _L + iota
            su = uidx[c, pl.ds(g * _L, _L)]
            si = iidx[c, pl.ds(g * _L, _L)]
            acc = (plsc.load_gather(ubb_c, [rowk, jnp.bitwise_and(su, 7)])
                   + plsc.load_gather(ibb_c, [rowk, jnp.bitwise_and(si, 7)]))
            col = jnp.full((_L,), 0, jnp.int32)
            for d in range(_D):
                cd = jnp.full((_L,), d, jnp.int32)
                pu = plsc.load_gather(ub_c, [rowk, cd])
                pi = plsc.load_gather(ib_c, [rowk, cd])
                acc = acc + pu * pi
            outv[pl.ds(c * _CHUNK + g * _L, _L)] = acc
            return carry

        lax.fori_loop(0, _GPC, body, 0)

    pltpu.sync_copy(outv, out_hbm.at[wid])


def kernel(user_ids, item_ids, user_emb, item_emb, user_bias, item_bias):
    uid = user_ids.astype(jnp.int32).reshape(_NW, _NCHUNK, _CHUNK)
    iid = item_ids.astype(jnp.int32).reshape(_NW, _NCHUNK, _CHUNK)
    ue = jnp.pad(user_emb, ((0, 0), (0, _D)))
    ie = jnp.pad(item_emb, ((0, 0), (0, _D)))
    ub = user_bias.reshape(-1, 8)
    ib = item_bias.reshape(-1, 8)
    out = _sc_kernel(uid, iid, ue, ie, ub, ib)
    return out.reshape(_B)
